# baseline (speedup 1.0000x reference)
import functools

import jax
import jax.numpy as jnp
from jax import lax
from jax.experimental import pallas as pl
from jax.experimental.pallas import tpu as pltpu
from jax.experimental.pallas import tpu_sc as plsc

B = 1024
S = 100000
T = 1000
T_PAD = 1024
CHUNK = 128
NUM_CHUNKS = T_PAD // CHUNK
NW = 32
ROWS_PER_W = B // NW
LANES = 16


def _sc_gather(flat_hbm, ysub_hbm, out_hbm, ysub_v, idx_v, row_v, sem):
    wid = lax.axis_index("s") * 2 + lax.axis_index("c")

    zeros = jnp.zeros((LANES,), jnp.int32)
    ysub_v[pl.ds(T, LANES)] = zeros
    ysub_v[pl.ds(T_PAD - LANES, LANES)] = zeros
    pltpu.sync_copy(ysub_hbm, ysub_v.at[pl.ds(0, T)])

    def row_body(r, carry):
        row = wid * ROWS_PER_W + r
        base = row * S
        for j in range(T_PAD // LANES):
            idx_v[pl.ds(j * LANES, LANES)] = ysub_v[pl.ds(j * LANES, LANES)] + base
        copies = []
        for c in range(NUM_CHUNKS):
            copies.append(
                pltpu.async_copy(
                    flat_hbm.at[idx_v.at[pl.ds(c * CHUNK, CHUNK)]],
                    row_v.at[pl.ds(c * CHUNK, CHUNK)],
                    sem,
                )
            )
        for cp in copies:
            cp.wait()
        pltpu.sync_copy(row_v.at[pl.ds(0, T)], out_hbm.at[pl.ds(row * T, T)])
        return carry

    lax.fori_loop(0, ROWS_PER_W, row_body, None)


def kernel(logits_p, y_sub):
    flat = logits_p.reshape(B * S)
    y32 = y_sub.astype(jnp.int32)
    mesh = plsc.VectorSubcoreMesh(core_axis_name="c", subcore_axis_name="s")
    f = functools.partial(
        pl.kernel,
        mesh=mesh,
        out_type=jax.ShapeDtypeStruct((B * T,), jnp.float32),
        scratch_types=[
            pltpu.VMEM((T_PAD,), jnp.int32),
            pltpu.VMEM((T_PAD,), jnp.int32),
            pltpu.VMEM((T_PAD,), jnp.float32),
            pltpu.SemaphoreType.DMA,
        ],
    )(_sc_gather)
    return f(flat, y32).reshape(B, T)
